# Initial kernel scaffold; baseline (speedup 1.0000x reference)
import jax
import jax.numpy as jnp
from jax.experimental import pallas as pl


def kernel(sample, item_id_table, category_table, category_map):
    raise NotImplementedError("write your pallas kernel here")


# speedup vs baseline: 3.3533x; 3.3533x over previous
import functools

import jax
import jax.numpy as jnp
from jax import lax
from jax.experimental import pallas as pl
from jax.experimental.pallas import tpu as pltpu
from jax.experimental.pallas import tpu_sc as plsc

_VOCAB = 1000000
_CAT_VOCAB = 100000
_ID_DIM = 64
_CAT_DIM = 64
_FINAL = _ID_DIM + _CAT_DIM
_B = 4096
_L = 200
_N = _B * _L

_NC = 2
_NS = 16
_NW = _NC * _NS
_PER_W = _N // _NW
_C = 512
_G = 128
_K = _C // _G
_CHUNKS = _PER_W // _C
_LANES = 16


def _body(flat_hbm, id_tab_hbm, cat_tab_hbm, cmap_hbm, out_hbm,
          idx_v, cidx_v, id_rows_v, cat_rows_v, sem_m, sem_a, sem_b):
    wid = lax.axis_index("s") * _NC + lax.axis_index("c")
    w_base = wid * _PER_W

    def chunk(i, carry):
        base = w_base + i * _C
        pltpu.sync_copy(flat_hbm.at[pl.ds(base, _C)], idx_v)

        for j in range(_K):
            pltpu.async_copy(
                cmap_hbm.at[idx_v.at[pl.ds(j * _G, _G)]],
                cidx_v.at[pl.ds(j * _G, _G)], sem_m).wait()

        def mask_step(t, carry):
            tok = idx_v[pl.ds(t * _LANES, _LANES)]
            cid = cidx_v[pl.ds(t * _LANES, _LANES)]
            cidx_v[pl.ds(t * _LANES, _LANES)] = jnp.where(
                tok == 0, jnp.zeros_like(cid), cid)
            return carry
        lax.fori_loop(0, _C // _LANES, mask_step, 0)

        copies = []
        for j in range(_K):
            copies.append(pltpu.async_copy(
                id_tab_hbm.at[idx_v.at[pl.ds(j * _G, _G)]],
                id_rows_v.at[pl.ds(j * _G, _G)], sem_a))
            copies.append(pltpu.async_copy(
                cat_tab_hbm.at[cidx_v.at[pl.ds(j * _G, _G)]],
                cat_rows_v.at[pl.ds(j * _G, _G)], sem_b))
        for c in copies:
            c.wait()

        pltpu.sync_copy(id_rows_v, out_hbm.at[pl.ds(base, _C), pl.ds(0, _ID_DIM)])
        pltpu.sync_copy(cat_rows_v, out_hbm.at[pl.ds(base, _C), pl.ds(_ID_DIM, _CAT_DIM)])
        return carry

    lax.fori_loop(0, _CHUNKS, chunk, 0)


@jax.jit
def _item_feat(flat, item_id_table, category_table, category_map):
    mesh = plsc.VectorSubcoreMesh(core_axis_name="c", subcore_axis_name="s")
    f = functools.partial(
        pl.kernel,
        out_type=jax.ShapeDtypeStruct((_N, _FINAL), jnp.float32),
        mesh=mesh,
        compiler_params=pltpu.CompilerParams(use_tc_tiling_on_sc=False),
        scratch_types=[
            pltpu.VMEM((_C,), jnp.int32),
            pltpu.VMEM((_C,), jnp.int32),
            pltpu.VMEM((_C, _ID_DIM), jnp.float32),
            pltpu.VMEM((_C, _CAT_DIM), jnp.float32),
            pltpu.SemaphoreType.DMA,
            pltpu.SemaphoreType.DMA,
            pltpu.SemaphoreType.DMA,
        ],
    )(_body)
    return f(flat, item_id_table, category_table, category_map)


def kernel(sample, item_id_table, category_table, category_map):
    flat = sample.reshape(-1)
    out = _item_feat(flat, item_id_table, category_table, category_map)
    return out.reshape(_B, _L, _FINAL)


# speedup vs baseline: 4.1657x; 1.2423x over previous
import functools

import jax
import jax.numpy as jnp
from jax import lax
from jax.experimental import pallas as pl
from jax.experimental.pallas import tpu as pltpu
from jax.experimental.pallas import tpu_sc as plsc

_ID_DIM = 64
_CAT_DIM = 64
_FINAL = _ID_DIM + _CAT_DIM
_B = 4096
_L = 200
_N = _B * _L

_NC = 2
_NS = 16
_NW = _NC * _NS
_PER_W = _N // _NW
_C = 256
_G = 128
_K = _C // _G
_CHUNKS = _PER_W // _C
_LANES = 16


def _body(flat_hbm, id_tab_hbm, cat_tab_hbm, cmap_hbm, out_hbm,
          idx_v, cidx_v, idr_v, catr_v, sem_cmap, sem_rows, sem_out):
    wid = lax.axis_index("s") * _NC + lax.axis_index("c")
    w_base = wid * _PER_W

    def out_slices(base):
        return (out_hbm.at[pl.ds(base, _C), pl.ds(0, _ID_DIM)],
                out_hbm.at[pl.ds(base, _C), pl.ds(_ID_DIM, _CAT_DIM)])

    def front(gc, b):
        base = w_base + gc * _C
        @pl.when(gc >= 2)
        def _():
            oid, ocat = out_slices(base - 2 * _C)
            pltpu.make_async_copy(oid, idr_v[b], sem_out[b]).wait()
            pltpu.make_async_copy(ocat, catr_v[b], sem_out[b]).wait()
        pltpu.sync_copy(flat_hbm.at[pl.ds(base, _C)], idx_v[b])
        for j in range(_K):
            sl = pl.ds(j * _G, _G)
            pltpu.async_copy(cmap_hbm.at[idx_v[b].at[sl]],
                             cidx_v[b].at[sl], sem_cmap[b])
            pltpu.async_copy(id_tab_hbm.at[idx_v[b].at[sl]],
                             idr_v[b].at[sl], sem_rows[b])

    def mid(gc, b):
        for j in range(_K):
            sl = pl.ds(j * _G, _G)
            pltpu.make_async_copy(cmap_hbm.at[idx_v[b].at[sl]],
                                  cidx_v[b].at[sl], sem_cmap[b]).wait()

        def mask_step(t, carry):
            sl = pl.ds(t * _LANES, _LANES)
            tok = idx_v[b][sl]
            cid = cidx_v[b][sl]
            cidx_v[b][sl] = jnp.where(tok == 0, jnp.zeros_like(cid), cid)
            return carry
        lax.fori_loop(0, _C // _LANES, mask_step, 0)

        for j in range(_K):
            sl = pl.ds(j * _G, _G)
            pltpu.async_copy(cat_tab_hbm.at[cidx_v[b].at[sl]],
                             catr_v[b].at[sl], sem_rows[b])

    def back(gc, b):
        base = w_base + gc * _C
        for j in range(_K):
            sl = pl.ds(j * _G, _G)
            pltpu.make_async_copy(id_tab_hbm.at[idx_v[b].at[sl]],
                                  idr_v[b].at[sl], sem_rows[b]).wait()
            pltpu.make_async_copy(cat_tab_hbm.at[cidx_v[b].at[sl]],
                                  catr_v[b].at[sl], sem_rows[b]).wait()
        oid, ocat = out_slices(base)
        pltpu.async_copy(idr_v[b], oid, sem_out[b])
        pltpu.async_copy(catr_v[b], ocat, sem_out[b])

    def pair(i, carry):
        g = 2 * i
        front(g, 0)
        front(g + 1, 1)
        mid(g, 0)
        mid(g + 1, 1)
        back(g, 0)
        back(g + 1, 1)
        return carry

    lax.fori_loop(0, _CHUNKS // 2, pair, 0)

    for b, gc in ((0, _CHUNKS - 2), (1, _CHUNKS - 1)):
        oid, ocat = out_slices(w_base + gc * _C)
        pltpu.make_async_copy(oid, idr_v[b], sem_out[b]).wait()
        pltpu.make_async_copy(ocat, catr_v[b], sem_out[b]).wait()


@jax.jit
def _item_feat(flat, item_id_table, category_table, category_map):
    mesh = plsc.VectorSubcoreMesh(core_axis_name="c", subcore_axis_name="s")
    f = functools.partial(
        pl.kernel,
        out_type=jax.ShapeDtypeStruct((_N, _FINAL), jnp.float32),
        mesh=mesh,
        compiler_params=pltpu.CompilerParams(use_tc_tiling_on_sc=False),
        scratch_types=[
            [pltpu.VMEM((_C,), jnp.int32)] * 2,
            [pltpu.VMEM((_C,), jnp.int32)] * 2,
            [pltpu.VMEM((_C, _ID_DIM), jnp.float32)] * 2,
            [pltpu.VMEM((_C, _CAT_DIM), jnp.float32)] * 2,
            [pltpu.SemaphoreType.DMA] * 2,
            [pltpu.SemaphoreType.DMA] * 2,
            [pltpu.SemaphoreType.DMA] * 2,
        ],
    )(_body)
    return f(flat, item_id_table, category_table, category_map)


def kernel(sample, item_id_table, category_table, category_map):
    flat = sample.reshape(-1)
    out = _item_feat(flat, item_id_table, category_table, category_map)
    return out.reshape(_B, _L, _FINAL)


# speedup vs baseline: 4.1905x; 1.0059x over previous
import functools

import jax
import jax.numpy as jnp
from jax import lax
from jax.experimental import pallas as pl
from jax.experimental.pallas import tpu as pltpu
from jax.experimental.pallas import tpu_sc as plsc

_ID_DIM = 64
_CAT_DIM = 64
_FINAL = _ID_DIM + _CAT_DIM
_B = 4096
_L = 200

_NC = 2
_NS = 16
_NW = _NC * _NS
_ROWS_W = _B // _NW
_NBUF = 4
_STEPS = _ROWS_W // _NBUF
_SPLITS = ((0, 104), (104, 96))
_MASK_OFFS = tuple(range(0, 192, 16)) + (184,)


def _body(sample_hbm, id_tab_hbm, cat_tab_hbm, cmap_hbm, out_hbm,
          idx_all, cidx_v, idr_v, catr_v, sem_cmap, sem_rows, sem_out):
    wid = lax.axis_index("s") * _NC + lax.axis_index("c")
    row0 = wid * _ROWS_W

    pltpu.sync_copy(sample_hbm.at[pl.ds(row0, _ROWS_W), :], idx_all)

    def out_slices(r):
        return (out_hbm.at[r, :, pl.ds(0, _ID_DIM)],
                out_hbm.at[r, :, pl.ds(_ID_DIM, _CAT_DIM)])

    def front(i, rl, b):
        @pl.when(i >= 1)
        def _():
            oid, ocat = out_slices(row0 + rl - _NBUF)
            pltpu.make_async_copy(oid, idr_v[b], sem_out[b]).wait()
            pltpu.make_async_copy(ocat, catr_v[b], sem_out[b]).wait()
        for off, g in _SPLITS:
            sl = pl.ds(off, g)
            pltpu.async_copy(cmap_hbm.at[idx_all.at[rl, sl]],
                             cidx_v[b].at[sl], sem_cmap[b])
            pltpu.async_copy(id_tab_hbm.at[idx_all.at[rl, sl]],
                             idr_v[b].at[sl], sem_rows[b])

    def mid(rl, b):
        for off, g in _SPLITS:
            sl = pl.ds(off, g)
            pltpu.make_async_copy(cmap_hbm.at[idx_all.at[rl, sl]],
                                  cidx_v[b].at[sl], sem_cmap[b]).wait()
        for o in _MASK_OFFS:
            sl = pl.ds(o, 16)
            tok = idx_all[rl, sl]
            cid = cidx_v[b][sl]
            cidx_v[b][sl] = jnp.where(tok == 0, jnp.zeros_like(cid), cid)
        for off, g in _SPLITS:
            sl = pl.ds(off, g)
            pltpu.async_copy(cat_tab_hbm.at[cidx_v[b].at[sl]],
                             catr_v[b].at[sl], sem_rows[b])

    def back(rl, b):
        for off, g in _SPLITS:
            sl = pl.ds(off, g)
            pltpu.make_async_copy(id_tab_hbm.at[idx_all.at[rl, sl]],
                                  idr_v[b].at[sl], sem_rows[b]).wait()
            pltpu.make_async_copy(cat_tab_hbm.at[cidx_v[b].at[sl]],
                                  catr_v[b].at[sl], sem_rows[b]).wait()
        oid, ocat = out_slices(row0 + rl)
        pltpu.async_copy(idr_v[b], oid, sem_out[b])
        pltpu.async_copy(catr_v[b], ocat, sem_out[b])

    def step(i, carry):
        for b in range(_NBUF):
            front(i, i * _NBUF + b, b)
        for b in range(_NBUF):
            mid(i * _NBUF + b, b)
        for b in range(_NBUF):
            back(i * _NBUF + b, b)
        return carry

    lax.fori_loop(0, _STEPS, step, 0)

    for b in range(_NBUF):
        oid, ocat = out_slices(row0 + _ROWS_W - _NBUF + b)
        pltpu.make_async_copy(oid, idr_v[b], sem_out[b]).wait()
        pltpu.make_async_copy(ocat, catr_v[b], sem_out[b]).wait()


@jax.jit
def kernel(sample, item_id_table, category_table, category_map):
    mesh = plsc.VectorSubcoreMesh(core_axis_name="c", subcore_axis_name="s")
    f = functools.partial(
        pl.kernel,
        out_type=jax.ShapeDtypeStruct((_B, _L, _FINAL), jnp.float32),
        mesh=mesh,
        compiler_params=pltpu.CompilerParams(use_tc_tiling_on_sc=False),
        scratch_types=[
            pltpu.VMEM((_ROWS_W, _L), jnp.int32),
            [pltpu.VMEM((_L,), jnp.int32)] * _NBUF,
            [pltpu.VMEM((_L, _ID_DIM), jnp.float32)] * _NBUF,
            [pltpu.VMEM((_L, _CAT_DIM), jnp.float32)] * _NBUF,
            [pltpu.SemaphoreType.DMA] * _NBUF,
            [pltpu.SemaphoreType.DMA] * _NBUF,
            [pltpu.SemaphoreType.DMA] * _NBUF,
        ],
    )(_body)
    return f(sample, item_id_table, category_table, category_map)
